# baseline (speedup 1.0000x reference)
import functools

import jax
import jax.numpy as jnp
from jax.experimental import pallas as pl
from jax.experimental.pallas import tpu as pltpu
from jax.experimental.pallas import tpu_sc as plsc

B = 1
S = 2048
D_MODEL = 1024
HIDDEN = 1024
OUT_DIM = 1024
NUM_EXPERTS = 8
TOP_K = 2

TILE = 256
NCHUNK = S // TILE
NT_ROUTED = (TOP_K * S + NUM_EXPERTS * (TILE - 1)) // TILE
CAP_R = NT_ROUTED * TILE
NT_SHARED = S // TILE
NT_TOTAL = NT_ROUTED + NT_SHARED
CAP_Y = NT_TOTAL * TILE

NPLANE = 4
PW = D_MODEL // NPLANE
SCW = 128
NW = S // SCW


@functools.cache
def _vector_mesh():
    return plsc.VectorSubcoreMesh(core_axis_name="c", subcore_axis_name="s")


def _router_kernel(x_ref, wr_ref, br_ref,
                   logits_ref, topk_ref, posb_ref, posd_ref, w_ref, te_ref,
                   xp_ref):
    xt = x_ref[...]
    logits = jnp.dot(xt, wr_ref[...], preferred_element_type=jnp.float32)
    logits = logits + br_ref[...]
    logits_ref[...] = logits
    m = jnp.max(logits, axis=1, keepdims=True)
    e = jnp.exp(logits - m)
    gw = e / jnp.sum(e, axis=1, keepdims=True)
    lane = jax.lax.broadcasted_iota(jnp.int32, (S, NUM_EXPERTS), 1)
    i1 = jnp.min(jnp.where(logits == m, lane, NUM_EXPERTS), axis=1,
                 keepdims=True)
    masked = jnp.where(lane == i1, -jnp.inf, logits)
    m2 = jnp.max(masked, axis=1, keepdims=True)
    i2 = jnp.min(jnp.where(masked == m2, lane, NUM_EXPERTS), axis=1,
                 keepdims=True)
    w1 = jnp.sum(jnp.where(lane == i1, gw, 0.0), axis=1, keepdims=True)
    w2 = jnp.sum(jnp.where(lane == i2, gw, 0.0), axis=1, keepdims=True)
    s = w1 + w2
    topk_ref[...] = jnp.concatenate([i1, i2], axis=1)
    w_ref[...] = jnp.concatenate([w1 / s, w2 / s], axis=1)

    memb = (jnp.where(lane == i1, 1.0, 0.0) + jnp.where(lane == i2, 1.0, 0.0))
    r_iota = jax.lax.broadcasted_iota(jnp.int32, (TILE, TILE), 0)
    c_iota = jax.lax.broadcasted_iota(jnp.int32, (TILE, TILE), 1)
    tril = jnp.where(r_iota > c_iota, 1.0, 0.0)
    running = jnp.zeros((1, NUM_EXPERTS), jnp.float32)
    ranks = []
    for c in range(NCHUNK):
        mc = memb[c * TILE:(c + 1) * TILE, :]
        ranks.append(jnp.dot(tril, mc, preferred_element_type=jnp.float32)
                     + running)
        running = running + jnp.sum(mc, axis=0, keepdims=True)
    rank = jnp.concatenate(ranks, axis=0)

    counts = running
    ci = counts.astype(jnp.int32)
    padded = ((ci + (TILE - 1)) // TILE) * TILE
    padded_f = padded.astype(jnp.float32)
    r8 = jax.lax.broadcasted_iota(jnp.int32, (NUM_EXPERTS, NUM_EXPERTS), 0)
    c8 = jax.lax.broadcasted_iota(jnp.int32, (NUM_EXPERTS, NUM_EXPERTS), 1)
    supper = jnp.where(r8 < c8, 1.0, 0.0)
    offs = jnp.dot(padded_f, supper, preferred_element_type=jnp.float32)

    offs_b = jnp.broadcast_to(offs, (S, NUM_EXPERTS))
    dest = offs_b + rank
    p1 = jnp.sum(jnp.where(lane == i1, dest, 0.0), axis=1, keepdims=True)
    p2 = jnp.sum(jnp.where(lane == i2, dest, 0.0), axis=1, keepdims=True)
    p1r = jnp.transpose(p1).astype(jnp.int32)
    p2r = jnp.transpose(p2).astype(jnp.int32)
    posb_ref[...] = jnp.concatenate(
        [p1r + k * CAP_R for k in range(NPLANE)]
        + [p2r + k * CAP_R for k in range(NPLANE)], axis=0)
    posd_ref[...] = jnp.concatenate(
        [p1r + k * CAP_Y for k in range(NPLANE)]
        + [p2r + k * CAP_Y for k in range(NPLANE)], axis=0)

    tl = jax.lax.broadcasted_iota(jnp.int32, (1, NT_TOTAL), 1)
    ts = (tl * TILE).astype(jnp.float32)
    acc = jnp.zeros((1, NT_TOTAL), jnp.float32)
    lane8 = jax.lax.broadcasted_iota(jnp.int32, (1, NUM_EXPERTS), 1)
    for ee in range(NUM_EXPERTS):
        off_e = jnp.sum(jnp.where(lane8 == ee, offs, 0.0))
        pad_e = jnp.sum(jnp.where(lane8 == ee, padded_f, 0.0))
        acc = acc + ee * jnp.where((ts >= off_e) & (ts < off_e + pad_e),
                                   1.0, 0.0)
    te = jnp.where(tl >= NT_ROUTED, NUM_EXPERTS, acc.astype(jnp.int32))
    te_ref[...] = te

    for k in range(NPLANE):
        xp_ref[k] = xt[:, k * PW:(k + 1) * PW]


def _router(x2, Wr, br):
    return pl.pallas_call(
        _router_kernel,
        out_shape=[
            jax.ShapeDtypeStruct((S, NUM_EXPERTS), jnp.float32),
            jax.ShapeDtypeStruct((S, TOP_K), jnp.int32),
            jax.ShapeDtypeStruct((TOP_K * NPLANE, S), jnp.int32),
            jax.ShapeDtypeStruct((TOP_K * NPLANE, S), jnp.int32),
            jax.ShapeDtypeStruct((S, TOP_K), jnp.float32),
            jax.ShapeDtypeStruct((1, NT_TOTAL), jnp.int32),
            jax.ShapeDtypeStruct((NPLANE, S, PW), jnp.float32),
        ],
    )(x2, Wr, br.reshape(1, NUM_EXPERTS))


def _sc_dispatch(xp, posb):
    @functools.partial(pl.kernel,
                       out_type=jax.ShapeDtypeStruct((NPLANE * CAP_R, PW),
                                                     jnp.float32),
                       mesh=_vector_mesh())
    def k(x_hbm, p_hbm, o_hbm):
        def body(x_vmem, i_vmem):
            pltpu.sync_copy(x_vmem.at[0], o_hbm.at[i_vmem.at[0]])
        pltpu.emit_pipeline(
            body,
            grid=(TOP_K, NPLANE * NW),
            in_specs=[pl.BlockSpec((1, SCW, PW),
                                   lambda s, ki: (ki // NW, ki % NW, 0)),
                      pl.BlockSpec((1, SCW),
                                   lambda s, ki: (s * NPLANE + ki // NW,
                                                  ki % NW))],
            out_specs=[],
            core_axis_name=("c", "s"),
            dimension_semantics=(pltpu.PARALLEL, pltpu.PARALLEL),
        )(x_hbm, p_hbm)
    return k(xp, posb)


def _sc_gather(y4, posd):
    @functools.partial(pl.kernel,
                       out_type=jax.ShapeDtypeStruct(
                           (NPLANE * TOP_K * S, PW), jnp.float32),
                       mesh=_vector_mesh())
    def k(y_hbm, p_hbm, o_hbm):
        def body(i_vmem, o_vmem):
            pltpu.sync_copy(y_hbm.at[i_vmem.at[0]], o_vmem)
        pltpu.emit_pipeline(
            body,
            grid=(TOP_K, NPLANE * NW),
            in_specs=[pl.BlockSpec((1, SCW),
                                   lambda s, ki: (s * NPLANE + ki // NW,
                                                  ki % NW))],
            out_specs=[pl.BlockSpec(
                (SCW, PW),
                lambda s, ki: ((ki // NW) * (TOP_K * NW) + s * NW + ki % NW,
                               0))],
            core_axis_name=("c", "s"),
            dimension_semantics=(pltpu.PARALLEL, pltpu.PARALLEL),
        )(p_hbm, o_hbm)
    return k(y4, posd)


def _gmm_kernel(te_ref, xd_ref, xs_ref, gw_ref, vw_ref, ow_ref, ob_ref,
                y_ref):
    j = pl.program_id(0)
    xd = jnp.concatenate([xd_ref[k] for k in range(NPLANE)], axis=1)
    xin = jnp.where(j < NT_ROUTED, xd, xs_ref[...]).astype(jnp.bfloat16)
    g = jnp.dot(xin, gw_ref[0], preferred_element_type=jnp.float32)
    v = jnp.dot(xin, vw_ref[0], preferred_element_type=jnp.float32)
    h = ((g * jax.lax.logistic(g)) * v).astype(jnp.bfloat16)
    y = jnp.dot(h, ow_ref[0], preferred_element_type=jnp.float32) + ob_ref[0]
    for k in range(NPLANE):
        y_ref[k] = y[:, k * PW:(k + 1) * PW]


def _gmm(te, x_disp, x2, gW, vW, oW, ob):
    grid_spec = pltpu.PrefetchScalarGridSpec(
        num_scalar_prefetch=1,
        grid=(NT_TOTAL,),
        in_specs=[
            pl.BlockSpec((NPLANE, TILE, PW),
                         lambda j, te: (0, jnp.minimum(j, NT_ROUTED - 1), 0)),
            pl.BlockSpec((TILE, D_MODEL),
                         lambda j, te: (jnp.maximum(j - NT_ROUTED, 0), 0)),
            pl.BlockSpec((1, D_MODEL, HIDDEN), lambda j, te: (te[j], 0, 0)),
            pl.BlockSpec((1, D_MODEL, HIDDEN), lambda j, te: (te[j], 0, 0)),
            pl.BlockSpec((1, HIDDEN, OUT_DIM), lambda j, te: (te[j], 0, 0)),
            pl.BlockSpec((1, 1, OUT_DIM), lambda j, te: (te[j], 0, 0)),
        ],
        out_specs=pl.BlockSpec((NPLANE, TILE, PW), lambda j, te: (0, j, 0)),
    )
    return pl.pallas_call(
        _gmm_kernel,
        grid_spec=grid_spec,
        out_shape=jax.ShapeDtypeStruct((NPLANE, CAP_Y, PW), jnp.float32),
    )(te, x_disp, x2, gW, vW, oW, ob)


def _combine_kernel(ysh_ref, y1_ref, y2_ref, w_ref, out_ref):
    w1 = w_ref[:, 0:1]
    w2 = w_ref[:, 1:2]
    ysh = jnp.concatenate([ysh_ref[k] for k in range(NPLANE)], axis=1)
    y1 = jnp.concatenate([y1_ref[k] for k in range(NPLANE)], axis=1)
    y2 = jnp.concatenate([y2_ref[k] for k in range(NPLANE)], axis=1)
    out_ref[...] = ysh + w1 * y1 + w2 * y2


def _combine(y4, y12, w):
    nb = S // TILE
    return pl.pallas_call(
        _combine_kernel,
        grid=(nb,),
        in_specs=[
            pl.BlockSpec((NPLANE, TILE, PW),
                         lambda i: (0, NT_ROUTED + i, 0)),
            pl.BlockSpec((NPLANE, TILE, PW), lambda i: (0, i, 0)),
            pl.BlockSpec((NPLANE, TILE, PW), lambda i: (0, nb + i, 0)),
            pl.BlockSpec((TILE, TOP_K), lambda i: (i, 0)),
        ],
        out_specs=pl.BlockSpec((TILE, OUT_DIM), lambda i: (i, 0)),
        out_shape=jax.ShapeDtypeStruct((S, OUT_DIM), jnp.float32),
    )(y4, y12, y12, w)


@jax.jit
def kernel(x, Wr, br, sgW, svW, soW, sob, egW, evW, eoW, eob):
    x2 = x.reshape(S, D_MODEL)
    gW_all = jnp.concatenate([egW, sgW[None]], axis=0).astype(jnp.bfloat16)
    vW_all = jnp.concatenate([evW, svW[None]], axis=0).astype(jnp.bfloat16)
    oW_all = jnp.concatenate([eoW, soW[None]], axis=0).astype(jnp.bfloat16)
    ob_all = jnp.concatenate([eob, sob[None]], axis=0).reshape(
        NUM_EXPERTS + 1, 1, OUT_DIM)

    logits, topk, posb, posd, w, te, xp = _router(x2, Wr, br)
    x_disp = _sc_dispatch(xp, posb).reshape(NPLANE, CAP_R, PW)
    y4 = _gmm(te.reshape(NT_TOTAL), x_disp, x2, gW_all, vW_all, oW_all,
              ob_all)
    y12 = _sc_gather(y4.reshape(NPLANE * CAP_Y, PW), posd).reshape(
        NPLANE, TOP_K * S, PW)
    out = _combine(y4, y12, w)

    return (out.reshape(B, S, OUT_DIM),
            logits.reshape(B, S, NUM_EXPERTS),
            topk.reshape(B, S, TOP_K))


# speedup vs baseline: 1.4087x; 1.4087x over previous
import functools

import jax
import jax.numpy as jnp
from jax.experimental import pallas as pl
from jax.experimental.pallas import tpu as pltpu
from jax.experimental.pallas import tpu_sc as plsc

B = 1
S = 2048
D_MODEL = 1024
HIDDEN = 1024
OUT_DIM = 1024
NUM_EXPERTS = 8
TOP_K = 2

TILE = 256
NCHUNK = S // TILE
NT_ROUTED = (TOP_K * S + NUM_EXPERTS * (TILE - 1)) // TILE
CAP_R = NT_ROUTED * TILE

NPLANE = 4
PW = D_MODEL // NPLANE
SCW = 128
NW = S // SCW
SH_TILE = 256


@functools.cache
def _vector_mesh():
    return plsc.VectorSubcoreMesh(core_axis_name="c", subcore_axis_name="s")


def _router_kernel(x_ref, wr_ref, br_ref,
                   logits_ref, topk_ref, posb_ref, w_ref, te_ref):
    xt = x_ref[...]
    logits = jnp.dot(xt, wr_ref[...], preferred_element_type=jnp.float32)
    logits = logits + br_ref[...]
    logits_ref[...] = logits
    m = jnp.max(logits, axis=1, keepdims=True)
    e = jnp.exp(logits - m)
    gw = e / jnp.sum(e, axis=1, keepdims=True)
    lane = jax.lax.broadcasted_iota(jnp.int32, (S, NUM_EXPERTS), 1)
    i1 = jnp.min(jnp.where(logits == m, lane, NUM_EXPERTS), axis=1,
                 keepdims=True)
    masked = jnp.where(lane == i1, -jnp.inf, logits)
    m2 = jnp.max(masked, axis=1, keepdims=True)
    i2 = jnp.min(jnp.where(masked == m2, lane, NUM_EXPERTS), axis=1,
                 keepdims=True)
    w1 = jnp.sum(jnp.where(lane == i1, gw, 0.0), axis=1, keepdims=True)
    w2 = jnp.sum(jnp.where(lane == i2, gw, 0.0), axis=1, keepdims=True)
    s = w1 + w2
    topk_ref[...] = jnp.concatenate([i1, i2], axis=1)
    w_ref[...] = jnp.concatenate([w1 / s, w2 / s], axis=1)

    memb = (jnp.where(lane == i1, 1.0, 0.0) + jnp.where(lane == i2, 1.0, 0.0))
    r_iota = jax.lax.broadcasted_iota(jnp.int32, (TILE, TILE), 0)
    c_iota = jax.lax.broadcasted_iota(jnp.int32, (TILE, TILE), 1)
    tril = jnp.where(r_iota > c_iota, 1.0, 0.0)
    running = jnp.zeros((1, NUM_EXPERTS), jnp.float32)
    ranks = []
    for c in range(NCHUNK):
        mc = memb[c * TILE:(c + 1) * TILE, :]
        ranks.append(jnp.dot(tril, mc, preferred_element_type=jnp.float32)
                     + running)
        running = running + jnp.sum(mc, axis=0, keepdims=True)
    rank = jnp.concatenate(ranks, axis=0)

    counts = running
    ci = counts.astype(jnp.int32)
    padded = ((ci + (TILE - 1)) // TILE) * TILE
    padded_f = padded.astype(jnp.float32)
    r8 = jax.lax.broadcasted_iota(jnp.int32, (NUM_EXPERTS, NUM_EXPERTS), 0)
    c8 = jax.lax.broadcasted_iota(jnp.int32, (NUM_EXPERTS, NUM_EXPERTS), 1)
    supper = jnp.where(r8 < c8, 1.0, 0.0)
    offs = jnp.dot(padded_f, supper, preferred_element_type=jnp.float32)

    offs_b = jnp.broadcast_to(offs, (S, NUM_EXPERTS))
    dest = offs_b + rank
    p1 = jnp.sum(jnp.where(lane == i1, dest, 0.0), axis=1, keepdims=True)
    p2 = jnp.sum(jnp.where(lane == i2, dest, 0.0), axis=1, keepdims=True)
    p1r = jnp.transpose(p1).astype(jnp.int32)
    p2r = jnp.transpose(p2).astype(jnp.int32)
    posb_ref[...] = jnp.concatenate(
        [p1r + k * CAP_R for k in range(NPLANE)]
        + [p2r + k * CAP_R for k in range(NPLANE)], axis=0)

    tl = jax.lax.broadcasted_iota(jnp.int32, (1, NT_ROUTED), 1)
    ts = (tl * TILE).astype(jnp.float32)
    acc = jnp.zeros((1, NT_ROUTED), jnp.float32)
    lane8 = jax.lax.broadcasted_iota(jnp.int32, (1, NUM_EXPERTS), 1)
    for ee in range(NUM_EXPERTS):
        off_e = jnp.sum(jnp.where(lane8 == ee, offs, 0.0))
        pad_e = jnp.sum(jnp.where(lane8 == ee, padded_f, 0.0))
        acc = acc + ee * jnp.where((ts >= off_e) & (ts < off_e + pad_e),
                                   1.0, 0.0)
    te_ref[...] = acc.astype(jnp.int32)


def _router(x2, Wr, br):
    return pl.pallas_call(
        _router_kernel,
        out_shape=[
            jax.ShapeDtypeStruct((S, NUM_EXPERTS), jnp.float32),
            jax.ShapeDtypeStruct((S, TOP_K), jnp.int32),
            jax.ShapeDtypeStruct((TOP_K * NPLANE, S), jnp.int32),
            jax.ShapeDtypeStruct((S, TOP_K), jnp.float32),
            jax.ShapeDtypeStruct((1, NT_ROUTED), jnp.int32),
        ],
    )(x2, Wr, br.reshape(1, NUM_EXPERTS))


def _sc_dispatch(x2, posb):
    @functools.partial(pl.kernel,
                       out_type=jax.ShapeDtypeStruct((NPLANE * CAP_R, PW),
                                                     jnp.float32),
                       mesh=_vector_mesh())
    def k(x_hbm, p_hbm, o_hbm):
        def body(x_vmem, i_vmem):
            pltpu.sync_copy(x_vmem, o_hbm.at[i_vmem.at[0]])
        pltpu.emit_pipeline(
            body,
            grid=(TOP_K, NPLANE * NW),
            in_specs=[pl.BlockSpec((SCW, PW),
                                   lambda s, ki: (ki % NW, ki // NW)),
                      pl.BlockSpec((1, SCW),
                                   lambda s, ki: (s * NPLANE + ki // NW,
                                                  ki % NW))],
            out_specs=[],
            core_axis_name=("c", "s"),
            dimension_semantics=(pltpu.PARALLEL, pltpu.PARALLEL),
        )(x_hbm, p_hbm)
    return k(x2, posb)


def _sc_gather(y4, posd):
    @functools.partial(pl.kernel,
                       out_type=jax.ShapeDtypeStruct(
                           (NPLANE * TOP_K * S, PW), jnp.float32),
                       mesh=_vector_mesh())
    def k(y_hbm, p_hbm, o_hbm):
        def body(i_vmem, o_vmem):
            pltpu.sync_copy(y_hbm.at[i_vmem.at[0]], o_vmem)
        pltpu.emit_pipeline(
            body,
            grid=(TOP_K, NPLANE * NW),
            in_specs=[pl.BlockSpec((1, SCW),
                                   lambda s, ki: (s * NPLANE + ki // NW,
                                                  ki % NW))],
            out_specs=[pl.BlockSpec(
                (SCW, PW),
                lambda s, ki: ((ki // NW) * (TOP_K * NW) + s * NW + ki % NW,
                               0))],
            core_axis_name=("c", "s"),
            dimension_semantics=(pltpu.PARALLEL, pltpu.PARALLEL),
        )(p_hbm, o_hbm)
    return k(y4, posd)


def _swiglu(xin, gw, vw, ow, ob):
    g = jnp.dot(xin, gw, preferred_element_type=jnp.float32)
    v = jnp.dot(xin, vw, preferred_element_type=jnp.float32)
    h = (g * jax.lax.logistic(g)) * v
    return jnp.dot(h, ow, preferred_element_type=jnp.float32) + ob


def _shared_kernel(x_ref, gw_ref, vw_ref, ow_ref, ob_ref, y_ref):
    y_ref[...] = _swiglu(x_ref[...], gw_ref[...], vw_ref[...], ow_ref[...],
                         ob_ref[...])


def _shared(x2, sgW, svW, soW, sob):
    nb = S // SH_TILE
    return pl.pallas_call(
        _shared_kernel,
        grid=(nb,),
        in_specs=[
            pl.BlockSpec((SH_TILE, D_MODEL), lambda i: (i, 0)),
            pl.BlockSpec((D_MODEL, HIDDEN), lambda i: (0, 0)),
            pl.BlockSpec((D_MODEL, HIDDEN), lambda i: (0, 0)),
            pl.BlockSpec((HIDDEN, OUT_DIM), lambda i: (0, 0)),
            pl.BlockSpec((1, OUT_DIM), lambda i: (0, 0)),
        ],
        out_specs=pl.BlockSpec((SH_TILE, OUT_DIM), lambda i: (i, 0)),
        out_shape=jax.ShapeDtypeStruct((S, OUT_DIM), jnp.float32),
    )(x2, sgW, svW, soW, sob.reshape(1, OUT_DIM))


def _gmm_kernel(te_ref, xd_ref, gw_ref, vw_ref, ow_ref, ob_ref, y_ref):
    xin = jnp.concatenate([xd_ref[k] for k in range(NPLANE)], axis=1)
    y = _swiglu(xin, gw_ref[0], vw_ref[0], ow_ref[0], ob_ref[0])
    for k in range(NPLANE):
        y_ref[k] = y[:, k * PW:(k + 1) * PW]


def _gmm(te, x_disp, egW, evW, eoW, eob):
    grid_spec = pltpu.PrefetchScalarGridSpec(
        num_scalar_prefetch=1,
        grid=(NT_ROUTED,),
        in_specs=[
            pl.BlockSpec((NPLANE, TILE, PW), lambda j, te: (0, j, 0)),
            pl.BlockSpec((1, D_MODEL, HIDDEN), lambda j, te: (te[j], 0, 0)),
            pl.BlockSpec((1, D_MODEL, HIDDEN), lambda j, te: (te[j], 0, 0)),
            pl.BlockSpec((1, HIDDEN, OUT_DIM), lambda j, te: (te[j], 0, 0)),
            pl.BlockSpec((1, 1, OUT_DIM), lambda j, te: (te[j], 0, 0)),
        ],
        out_specs=pl.BlockSpec((NPLANE, TILE, PW), lambda j, te: (0, j, 0)),
    )
    return pl.pallas_call(
        _gmm_kernel,
        grid_spec=grid_spec,
        out_shape=jax.ShapeDtypeStruct((NPLANE, CAP_R, PW), jnp.float32),
    )(te, x_disp, egW, evW, eoW, eob)


def _combine_kernel(ysh_ref, y1_ref, y2_ref, w_ref, out_ref):
    w1 = w_ref[:, 0:1]
    w2 = w_ref[:, 1:2]
    y1 = jnp.concatenate([y1_ref[k] for k in range(NPLANE)], axis=1)
    y2 = jnp.concatenate([y2_ref[k] for k in range(NPLANE)], axis=1)
    out_ref[...] = ysh_ref[...] + w1 * y1 + w2 * y2


def _combine(ysh, y12, w):
    nb = S // TILE
    return pl.pallas_call(
        _combine_kernel,
        grid=(nb,),
        in_specs=[
            pl.BlockSpec((TILE, OUT_DIM), lambda i: (i, 0)),
            pl.BlockSpec((NPLANE, TILE, PW), lambda i: (0, i, 0)),
            pl.BlockSpec((NPLANE, TILE, PW), lambda i: (0, nb + i, 0)),
            pl.BlockSpec((TILE, TOP_K), lambda i: (i, 0)),
        ],
        out_specs=pl.BlockSpec((TILE, OUT_DIM), lambda i: (i, 0)),
        out_shape=jax.ShapeDtypeStruct((S, OUT_DIM), jnp.float32),
    )(ysh, y12, y12, w)


@jax.jit
def kernel(x, Wr, br, sgW, svW, soW, sob, egW, evW, eoW, eob):
    x2 = x.reshape(S, D_MODEL)

    logits, topk, posb, w, te = _router(x2, Wr, br)
    x_disp = _sc_dispatch(x2, posb).reshape(NPLANE, CAP_R, PW)
    ysh = _shared(x2, sgW, svW, soW, sob)
    y4 = _gmm(te.reshape(NT_ROUTED), x_disp, egW, evW, eoW,
              eob.reshape(NUM_EXPERTS, 1, OUT_DIM))
    y12 = _sc_gather(y4.reshape(NPLANE * CAP_R, PW), posb).reshape(
        NPLANE, TOP_K * S, PW)
    out = _combine(ysh, y12, w)

    return (out.reshape(B, S, OUT_DIM),
            logits.reshape(B, S, NUM_EXPERTS),
            topk.reshape(B, S, TOP_K))


# speedup vs baseline: 1.4216x; 1.0091x over previous
import functools

import jax
import jax.numpy as jnp
from jax.experimental import pallas as pl
from jax.experimental.pallas import tpu as pltpu
from jax.experimental.pallas import tpu_sc as plsc

B = 1
S = 2048
D_MODEL = 1024
HIDDEN = 1024
OUT_DIM = 1024
NUM_EXPERTS = 8
TOP_K = 2

TILE = 256
NCHUNK = S // TILE
NT_ROUTED = (TOP_K * S + NUM_EXPERTS * (TILE - 1)) // TILE
CAP_R = NT_ROUTED * TILE

NPLANE = 4
PW = D_MODEL // NPLANE
SCW = 128
NW = S // SCW
SH_TILE = 256


@functools.cache
def _vector_mesh():
    return plsc.VectorSubcoreMesh(core_axis_name="c", subcore_axis_name="s")


def _router_kernel(x_ref, wr_ref, br_ref,
                   logits_ref, topk_ref, posb_ref, w_ref, te_ref):
    xt = x_ref[...]
    logits = jnp.dot(xt, wr_ref[...], preferred_element_type=jnp.float32)
    logits = logits + br_ref[...]
    logits_ref[...] = logits
    m = jnp.max(logits, axis=1, keepdims=True)
    e = jnp.exp(logits - m)
    gw = e / jnp.sum(e, axis=1, keepdims=True)
    lane = jax.lax.broadcasted_iota(jnp.int32, (S, NUM_EXPERTS), 1)
    i1 = jnp.min(jnp.where(logits == m, lane, NUM_EXPERTS), axis=1,
                 keepdims=True)
    masked = jnp.where(lane == i1, -jnp.inf, logits)
    m2 = jnp.max(masked, axis=1, keepdims=True)
    i2 = jnp.min(jnp.where(masked == m2, lane, NUM_EXPERTS), axis=1,
                 keepdims=True)
    w1 = jnp.sum(jnp.where(lane == i1, gw, 0.0), axis=1, keepdims=True)
    w2 = jnp.sum(jnp.where(lane == i2, gw, 0.0), axis=1, keepdims=True)
    s = w1 + w2
    topk_ref[...] = jnp.concatenate([i1, i2], axis=1)
    w_ref[...] = jnp.concatenate([w1 / s, w2 / s], axis=1)

    memb = (jnp.where(lane == i1, 1.0, 0.0) + jnp.where(lane == i2, 1.0, 0.0))
    r_iota = jax.lax.broadcasted_iota(jnp.int32, (TILE, TILE), 0)
    c_iota = jax.lax.broadcasted_iota(jnp.int32, (TILE, TILE), 1)
    tril = jnp.where(r_iota > c_iota, 1.0, 0.0)
    running = jnp.zeros((1, NUM_EXPERTS), jnp.float32)
    ranks = []
    for c in range(NCHUNK):
        mc = memb[c * TILE:(c + 1) * TILE, :]
        ranks.append(jnp.dot(tril, mc, preferred_element_type=jnp.float32)
                     + running)
        running = running + jnp.sum(mc, axis=0, keepdims=True)
    rank = jnp.concatenate(ranks, axis=0)

    counts = running
    ci = counts.astype(jnp.int32)
    padded = ((ci + (TILE - 1)) // TILE) * TILE
    padded_f = padded.astype(jnp.float32)
    r8 = jax.lax.broadcasted_iota(jnp.int32, (NUM_EXPERTS, NUM_EXPERTS), 0)
    c8 = jax.lax.broadcasted_iota(jnp.int32, (NUM_EXPERTS, NUM_EXPERTS), 1)
    supper = jnp.where(r8 < c8, 1.0, 0.0)
    offs = jnp.dot(padded_f, supper, preferred_element_type=jnp.float32)

    offs_b = jnp.broadcast_to(offs, (S, NUM_EXPERTS))
    dest = offs_b + rank
    p1 = jnp.sum(jnp.where(lane == i1, dest, 0.0), axis=1, keepdims=True)
    p2 = jnp.sum(jnp.where(lane == i2, dest, 0.0), axis=1, keepdims=True)
    p1r = jnp.transpose(p1).astype(jnp.int32)
    p2r = jnp.transpose(p2).astype(jnp.int32)
    posb_ref[...] = jnp.concatenate(
        [p1r + k * CAP_R for k in range(NPLANE)]
        + [p2r + k * CAP_R for k in range(NPLANE)], axis=0)

    tl = jax.lax.broadcasted_iota(jnp.int32, (1, NT_ROUTED), 1)
    ts = (tl * TILE).astype(jnp.float32)
    acc = jnp.zeros((1, NT_ROUTED), jnp.float32)
    lane8 = jax.lax.broadcasted_iota(jnp.int32, (1, NUM_EXPERTS), 1)
    for ee in range(NUM_EXPERTS):
        off_e = jnp.sum(jnp.where(lane8 == ee, offs, 0.0))
        pad_e = jnp.sum(jnp.where(lane8 == ee, padded_f, 0.0))
        acc = acc + ee * jnp.where((ts >= off_e) & (ts < off_e + pad_e),
                                   1.0, 0.0)
    te_ref[...] = acc.astype(jnp.int32)


def _router(x2, Wr, br):
    return pl.pallas_call(
        _router_kernel,
        out_shape=[
            jax.ShapeDtypeStruct((S, NUM_EXPERTS), jnp.float32),
            jax.ShapeDtypeStruct((S, TOP_K), jnp.int32),
            jax.ShapeDtypeStruct((TOP_K * NPLANE, S), jnp.int32),
            jax.ShapeDtypeStruct((S, TOP_K), jnp.float32),
            jax.ShapeDtypeStruct((1, NT_ROUTED), jnp.int32),
        ],
    )(x2, Wr, br.reshape(1, NUM_EXPERTS))


def _sc_dispatch(x2, posb):
    @functools.partial(pl.kernel,
                       out_type=jax.ShapeDtypeStruct((NPLANE * CAP_R, PW),
                                                     jnp.float32),
                       mesh=_vector_mesh())
    def k(x_hbm, p_hbm, o_hbm):
        def body(x_vmem, i_vmem):
            pltpu.sync_copy(x_vmem, o_hbm.at[i_vmem.at[0]])
        pltpu.emit_pipeline(
            body,
            grid=(TOP_K, NPLANE * NW),
            in_specs=[pl.BlockSpec((SCW, PW),
                                   lambda s, ki: (ki % NW, ki // NW)),
                      pl.BlockSpec((1, SCW),
                                   lambda s, ki: (s * NPLANE + ki // NW,
                                                  ki % NW))],
            out_specs=[],
            core_axis_name=("c", "s"),
            dimension_semantics=(pltpu.PARALLEL, pltpu.PARALLEL),
        )(x_hbm, p_hbm)
    return k(x2, posb)


def _sc_gather(y4, posd):
    @functools.partial(pl.kernel,
                       out_type=jax.ShapeDtypeStruct(
                           (NPLANE * TOP_K * S, PW), jnp.float32),
                       mesh=_vector_mesh())
    def k(y_hbm, p_hbm, o_hbm):
        def body(i_vmem, o_vmem):
            pltpu.sync_copy(y_hbm.at[i_vmem.at[0]], o_vmem)
        pltpu.emit_pipeline(
            body,
            grid=(TOP_K, NPLANE * NW),
            in_specs=[pl.BlockSpec((1, SCW),
                                   lambda s, ki: (s * NPLANE + ki // NW,
                                                  ki % NW))],
            out_specs=[pl.BlockSpec(
                (SCW, PW),
                lambda s, ki: ((ki // NW) * (TOP_K * NW) + s * NW + ki % NW,
                               0))],
            core_axis_name=("c", "s"),
            dimension_semantics=(pltpu.PARALLEL, pltpu.PARALLEL),
        )(p_hbm, o_hbm)
    return k(y4, posd)


def _swiglu(xin, gw, vw, ow, ob):
    g = jnp.dot(xin, gw, preferred_element_type=jnp.float32)
    v = jnp.dot(xin, vw, preferred_element_type=jnp.float32)
    h = (g * jax.lax.logistic(g)) * v
    return jnp.dot(h, ow, preferred_element_type=jnp.float32) + ob


def _shared_kernel(x_ref, gw_ref, vw_ref, ow_ref, ob_ref, y_ref):
    y_ref[...] = _swiglu(x_ref[...], gw_ref[...], vw_ref[...], ow_ref[...],
                         ob_ref[...])


def _shared(x2, sgW, svW, soW, sob):
    nb = S // SH_TILE
    return pl.pallas_call(
        _shared_kernel,
        grid=(nb,),
        in_specs=[
            pl.BlockSpec((SH_TILE, D_MODEL), lambda i: (i, 0)),
            pl.BlockSpec((D_MODEL, HIDDEN), lambda i: (0, 0)),
            pl.BlockSpec((D_MODEL, HIDDEN), lambda i: (0, 0)),
            pl.BlockSpec((HIDDEN, OUT_DIM), lambda i: (0, 0)),
            pl.BlockSpec((1, OUT_DIM), lambda i: (0, 0)),
        ],
        out_specs=pl.BlockSpec((SH_TILE, OUT_DIM), lambda i: (i, 0)),
        out_shape=jax.ShapeDtypeStruct((S, OUT_DIM), jnp.float32),
        compiler_params=pltpu.CompilerParams(
            dimension_semantics=("parallel",)),
    )(x2, sgW, svW, soW, sob.reshape(1, OUT_DIM))


def _gmm_kernel(te_ref, xd_ref, gw_ref, vw_ref, ow_ref, ob_ref, y_ref):
    xin = jnp.concatenate([xd_ref[k] for k in range(NPLANE)], axis=1)
    y = _swiglu(xin, gw_ref[0], vw_ref[0], ow_ref[0], ob_ref[0])
    for k in range(NPLANE):
        y_ref[k] = y[:, k * PW:(k + 1) * PW]


def _gmm(te, x_disp, egW, evW, eoW, eob):
    grid_spec = pltpu.PrefetchScalarGridSpec(
        num_scalar_prefetch=1,
        grid=(NT_ROUTED,),
        in_specs=[
            pl.BlockSpec((NPLANE, TILE, PW), lambda j, te: (0, j, 0)),
            pl.BlockSpec((1, D_MODEL, HIDDEN), lambda j, te: (te[j], 0, 0)),
            pl.BlockSpec((1, D_MODEL, HIDDEN), lambda j, te: (te[j], 0, 0)),
            pl.BlockSpec((1, HIDDEN, OUT_DIM), lambda j, te: (te[j], 0, 0)),
            pl.BlockSpec((1, 1, OUT_DIM), lambda j, te: (te[j], 0, 0)),
        ],
        out_specs=pl.BlockSpec((NPLANE, TILE, PW), lambda j, te: (0, j, 0)),
    )
    return pl.pallas_call(
        _gmm_kernel,
        grid_spec=grid_spec,
        out_shape=jax.ShapeDtypeStruct((NPLANE, CAP_R, PW), jnp.float32),
        compiler_params=pltpu.CompilerParams(
            dimension_semantics=("parallel",)),
    )(te, x_disp, egW, evW, eoW, eob)


def _combine_kernel(ysh_ref, y1_ref, y2_ref, w_ref, out_ref):
    w1 = w_ref[:, 0:1]
    w2 = w_ref[:, 1:2]
    y1 = jnp.concatenate([y1_ref[k] for k in range(NPLANE)], axis=1)
    y2 = jnp.concatenate([y2_ref[k] for k in range(NPLANE)], axis=1)
    out_ref[...] = ysh_ref[...] + w1 * y1 + w2 * y2


def _combine(ysh, y12, w):
    nb = S // TILE
    return pl.pallas_call(
        _combine_kernel,
        grid=(nb,),
        in_specs=[
            pl.BlockSpec((TILE, OUT_DIM), lambda i: (i, 0)),
            pl.BlockSpec((NPLANE, TILE, PW), lambda i: (0, i, 0)),
            pl.BlockSpec((NPLANE, TILE, PW), lambda i: (0, nb + i, 0)),
            pl.BlockSpec((TILE, TOP_K), lambda i: (i, 0)),
        ],
        out_specs=pl.BlockSpec((TILE, OUT_DIM), lambda i: (i, 0)),
        out_shape=jax.ShapeDtypeStruct((S, OUT_DIM), jnp.float32),
        compiler_params=pltpu.CompilerParams(
            dimension_semantics=("parallel",)),
    )(ysh, y12, y12, w)


@jax.jit
def kernel(x, Wr, br, sgW, svW, soW, sob, egW, evW, eoW, eob):
    x2 = x.reshape(S, D_MODEL)

    logits, topk, posb, w, te = _router(x2, Wr, br)
    x_disp = _sc_dispatch(x2, posb).reshape(NPLANE, CAP_R, PW)
    ysh = _shared(x2, sgW, svW, soW, sob)
    y4 = _gmm(te.reshape(NT_ROUTED), x_disp, egW, evW, eoW,
              eob.reshape(NUM_EXPERTS, 1, OUT_DIM))
    y12 = _sc_gather(y4.reshape(NPLANE * CAP_R, PW), posb).reshape(
        NPLANE, TOP_K * S, PW)
    out = _combine(ysh, y12, w)

    return (out.reshape(B, S, OUT_DIM),
            logits.reshape(B, S, NUM_EXPERTS),
            topk.reshape(B, S, TOP_K))
